# baseline (speedup 1.0000x reference)
import functools

import jax
import jax.numpy as jnp
from jax import lax
from jax.experimental import pallas as pl
from jax.experimental.pallas import tpu as pltpu
from jax.experimental.pallas import tpu_sc as plsc

NC = 2
NS = 16
NW = NC * NS
B = 128


def _sc_degree(dst2d, n_pad, k):
    rpt = n_pad // NS

    mesh = plsc.VectorSubcoreMesh(core_axis_name="c", subcore_axis_name="s")

    @functools.partial(
        pl.kernel,
        out_type=jax.ShapeDtypeStruct((NC, n_pad), jnp.float32),
        mesh=mesh,
        compiler_params=pltpu.CompilerParams(use_tc_tiling_on_sc=False),
        scratch_types=[
            pltpu.VMEM((k, B), jnp.int32),
            pltpu.VMEM((B,), jnp.float32),
            pltpu.VMEM((B,), jnp.float32),
            pltpu.VMEM_SHARED((n_pad,), jnp.float32),
            pltpu.SemaphoreType.DMA,
        ],
    )
    def deg_kernel(dst_hbm, deg_out, dstv, onesv, zerov, deg_sh, dsem):
        c = lax.axis_index("c")
        s = lax.axis_index("s")
        w = c * NS + s
        for i in range(B // 16):
            onesv[pl.ds(i * 16, 16)] = jnp.full((16,), 1.0, jnp.float32)
            zerov[pl.ds(i * 16, 16)] = jnp.zeros((16,), jnp.float32)

        def zbody(j, carry):
            pltpu.sync_copy(zerov, deg_sh.at[pl.ds(s * rpt + j * B, B)])
            return carry

        lax.fori_loop(0, rpt // B, zbody, 0)
        pltpu.sync_copy(dst_hbm.at[pl.ds(w * k, k)], dstv)
        plsc.subcore_barrier()

        def body(j, carry):
            pltpu.async_copy(onesv, deg_sh.at[dstv.at[j]], dsem, add=True)
            return carry

        lax.fori_loop(0, k, body, 0)

        def dbody(j, carry):
            pltpu.make_async_copy(onesv, deg_sh.at[dstv.at[j]], dsem).wait()
            return carry

        lax.fori_loop(0, k, dbody, 0)
        plsc.subcore_barrier()
        pltpu.sync_copy(deg_sh.at[pl.ds(s * rpt, rpt)],
                        deg_out.at[c, pl.ds(s * rpt, rpt)])

    return deg_kernel(dst2d)


def _sc_aggregate(y, src1d, dst2d, n_pad, k):
    ept = k * B
    k2 = k // 2
    ept2 = k2 * B
    rpt = n_pad // NS
    d = y.shape[1]

    mesh = plsc.VectorSubcoreMesh(core_axis_name="c", subcore_axis_name="s")

    @functools.partial(
        pl.kernel,
        out_type=jax.ShapeDtypeStruct((NC, n_pad, d), jnp.float32),
        mesh=mesh,
        compiler_params=pltpu.CompilerParams(use_tc_tiling_on_sc=False),
        scratch_types=[
            pltpu.VMEM((ept2,), jnp.int32),
            pltpu.VMEM((k2, B), jnp.int32),
            pltpu.VMEM((B, d), jnp.float32),
            pltpu.VMEM((B, d), jnp.float32),
            pltpu.VMEM_SHARED((n_pad, d), jnp.float32),
            pltpu.SemaphoreType.DMA,
            pltpu.SemaphoreType.DMA,
        ],
    )
    def agg_kernel(y_hbm, src_hbm, dst_hbm, out_hbm,
                   srcv, dstv, rows0, rows1, acc_sh, sem0, sem1):
        c = lax.axis_index("c")
        s = lax.axis_index("s")
        w = c * NS + s

        def zrow(i, carry):
            for j in range(d // 16):
                rows0[i, pl.ds(j * 16, 16)] = jnp.zeros((16,), jnp.float32)
            return carry

        lax.fori_loop(0, B, zrow, 0)

        def zbody(j, carry):
            pltpu.async_copy(rows0, acc_sh.at[pl.ds(s * rpt + j * B, B)],
                             sem0)
            return carry

        lax.fori_loop(0, rpt // B, zbody, 0)

        def zdrain(j, carry):
            pltpu.make_async_copy(
                rows0, acc_sh.at[pl.ds(s * rpt + j * B, B)], sem0).wait()
            return carry

        lax.fori_loop(0, rpt // B, zdrain, 0)
        plsc.subcore_barrier()

        for h in range(2):
            pltpu.sync_copy(src_hbm.at[pl.ds(w * ept + h * ept2, ept2)],
                            srcv)
            pltpu.sync_copy(dst_hbm.at[pl.ds(w * k + h * k2, k2)], dstv)
            pltpu.async_copy(y_hbm.at[srcv.at[pl.ds(0, B)]], rows0, sem0)

            def body(i, carry):
                j0 = 2 * i
                pltpu.async_copy(
                    y_hbm.at[srcv.at[pl.ds((j0 + 1) * B, B)]], rows1, sem1)
                pltpu.make_async_copy(
                    y_hbm.at[srcv.at[pl.ds(j0 * B, B)]], rows0, sem0).wait()
                pltpu.sync_copy(rows0, acc_sh.at[dstv.at[j0]], add=True)

                @pl.when(i + 1 < k2 // 2)
                def _():
                    pltpu.async_copy(
                        y_hbm.at[srcv.at[pl.ds((j0 + 2) * B, B)]], rows0,
                        sem0)

                pltpu.make_async_copy(
                    y_hbm.at[srcv.at[pl.ds((j0 + 1) * B, B)]], rows1,
                    sem1).wait()
                pltpu.sync_copy(rows1, acc_sh.at[dstv.at[j0 + 1]], add=True)
                return carry

            lax.fori_loop(0, k2 // 2, body, 0)

        plsc.subcore_barrier()
        pltpu.sync_copy(acc_sh.at[pl.ds(s * rpt, rpt)],
                        out_hbm.at[c, pl.ds(s * rpt, rpt)])

    return agg_kernel(y, src1d, dst2d)


def _tc_scale_matmul(x, w1, dega, degb, blk):
    n, d_in = x.shape
    d_out = w1.shape[1]
    grid = (n // blk,)

    def body(x_ref, w_ref, a_ref, b_ref, y_ref):
        dis = lax.rsqrt(1.0 + a_ref[...] + b_ref[...])
        y_ref[...] = jnp.dot(x_ref[...], w_ref[...],
                             preferred_element_type=jnp.float32) * dis

    return pl.pallas_call(
        body,
        grid=grid,
        in_specs=[
            pl.BlockSpec((blk, d_in), lambda i: (i, 0)),
            pl.BlockSpec((d_in, d_out), lambda i: (0, 0)),
            pl.BlockSpec((blk, 1), lambda i: (i, 0)),
            pl.BlockSpec((blk, 1), lambda i: (i, 0)),
        ],
        out_specs=pl.BlockSpec((blk, d_out), lambda i: (i, 0)),
        out_shape=jax.ShapeDtypeStruct((n, d_out), jnp.float32),
    )(x, w1, dega, degb)


def _tc_epilogue_matmul(acc, y1, dega, degb, b1, w2, blk):
    n, d = y1.shape
    d_out = w2.shape[1]
    grid = (n // blk,)

    def body(acc_ref, y1_ref, a_ref, b_ref, b1_ref, w_ref, y2_ref):
        dis = lax.rsqrt(1.0 + a_ref[...] + b_ref[...])
        h = (acc_ref[0] + acc_ref[1] + y1_ref[...]) * dis + b1_ref[...]
        h = jnp.maximum(h, 0.0)
        y2_ref[...] = jnp.dot(h, w_ref[...],
                              preferred_element_type=jnp.float32) * dis

    return pl.pallas_call(
        body,
        grid=grid,
        in_specs=[
            pl.BlockSpec((NC, blk, d), lambda i: (0, i, 0)),
            pl.BlockSpec((blk, d), lambda i: (i, 0)),
            pl.BlockSpec((blk, 1), lambda i: (i, 0)),
            pl.BlockSpec((blk, 1), lambda i: (i, 0)),
            pl.BlockSpec((1, d), lambda i: (0, 0)),
            pl.BlockSpec((d, d_out), lambda i: (0, 0)),
        ],
        out_specs=pl.BlockSpec((blk, d_out), lambda i: (i, 0)),
        out_shape=jax.ShapeDtypeStruct((n, d_out), jnp.float32),
    )(acc, y1, dega, degb, b1, w2)


def _tc_final(acc, y2, dega, degb, b2, blk):
    n, d = y2.shape
    grid = (n // blk,)

    def body(acc_ref, y2_ref, a_ref, b_ref, b2_ref, o_ref):
        dis = lax.rsqrt(1.0 + a_ref[...] + b_ref[...])
        o_ref[...] = (acc_ref[0] + acc_ref[1] + y2_ref[...]) * dis \
            + b2_ref[...]

    return pl.pallas_call(
        body,
        grid=grid,
        in_specs=[
            pl.BlockSpec((NC, blk, d), lambda i: (0, i, 0)),
            pl.BlockSpec((blk, d), lambda i: (i, 0)),
            pl.BlockSpec((blk, 1), lambda i: (i, 0)),
            pl.BlockSpec((blk, 1), lambda i: (i, 0)),
            pl.BlockSpec((1, d), lambda i: (0, 0)),
        ],
        out_specs=pl.BlockSpec((blk, d), lambda i: (i, 0)),
        out_shape=jax.ShapeDtypeStruct((n, d), jnp.float32),
    )(acc, y2, dega, degb, b2)


def kernel(x, edge_index, W1, b1, W2, b2):
    n, d_in = x.shape
    e = edge_index.shape[1]

    k = -(-e // (NW * B * 8)) * 8
    e_pad = NW * k * B
    n_pad = -(-(n + 1) // (B * NS)) * (B * NS)
    n_dummy = n_pad - n

    src = edge_index[0].astype(jnp.int32)
    dst = edge_index[1].astype(jnp.int32)
    npe = e_pad - e
    pad_idx = jnp.arange(npe, dtype=jnp.int32)
    src1d = jnp.concatenate([src, (pad_idx * 997) % n])
    dst2d = jnp.concatenate([dst, n + pad_idx % n_dummy]).reshape(NW * k, B)

    degp = _sc_degree(dst2d, n_pad, k)
    dega = degp[0].reshape(n_pad, 1)
    degb = degp[1].reshape(n_pad, 1)

    blk = 1000
    y1 = _tc_scale_matmul(x, W1, dega, degb, blk)
    acc1 = _sc_aggregate(y1, src1d, dst2d, n_pad, k)
    y2 = _tc_epilogue_matmul(acc1, y1, dega, degb, b1.reshape(1, -1), W2,
                             blk)
    acc2 = _sc_aggregate(y2, src1d, dst2d, n_pad, k)
    out = _tc_final(acc2, y2, dega, degb, b2.reshape(1, -1), blk)
    return out


# speedup vs baseline: 1.0350x; 1.0350x over previous
import functools

import jax
import jax.numpy as jnp
from jax import lax
from jax.experimental import pallas as pl
from jax.experimental.pallas import tpu as pltpu
from jax.experimental.pallas import tpu_sc as plsc

NC = 2
NS = 16
NW = NC * NS
B = 128


def _sc_degree(dst2d, n_pad, k):
    rpt = n_pad // NS

    mesh = plsc.VectorSubcoreMesh(core_axis_name="c", subcore_axis_name="s")

    @functools.partial(
        pl.kernel,
        out_type=jax.ShapeDtypeStruct((NC, n_pad), jnp.float32),
        mesh=mesh,
        compiler_params=pltpu.CompilerParams(use_tc_tiling_on_sc=False),
        scratch_types=[
            pltpu.VMEM((k, B), jnp.int32),
            pltpu.VMEM((B,), jnp.float32),
            pltpu.VMEM((B,), jnp.float32),
            pltpu.VMEM_SHARED((n_pad,), jnp.float32),
            pltpu.SemaphoreType.DMA,
        ],
    )
    def deg_kernel(dst_hbm, deg_out, dstv, onesv, zerov, deg_sh, dsem):
        c = lax.axis_index("c")
        s = lax.axis_index("s")
        w = c * NS + s
        for i in range(B // 16):
            onesv[pl.ds(i * 16, 16)] = jnp.full((16,), 1.0, jnp.float32)
            zerov[pl.ds(i * 16, 16)] = jnp.zeros((16,), jnp.float32)

        def zbody(j, carry):
            pltpu.sync_copy(zerov, deg_sh.at[pl.ds(s * rpt + j * B, B)])
            return carry

        lax.fori_loop(0, rpt // B, zbody, 0)
        pltpu.sync_copy(dst_hbm.at[pl.ds(w * k, k)], dstv)
        plsc.subcore_barrier()

        def body(j, carry):
            pltpu.async_copy(onesv, deg_sh.at[dstv.at[j]], dsem, add=True)
            return carry

        lax.fori_loop(0, k, body, 0)

        def dbody(j, carry):
            pltpu.make_async_copy(onesv, deg_sh.at[dstv.at[j]], dsem).wait()
            return carry

        lax.fori_loop(0, k, dbody, 0)
        plsc.subcore_barrier()
        pltpu.sync_copy(deg_sh.at[pl.ds(s * rpt, rpt)],
                        deg_out.at[c, pl.ds(s * rpt, rpt)])

    return deg_kernel(dst2d)


def _sc_aggregate(y, src1d, dst2d, n_pad, k):
    ept = k * B
    k2 = k // 2
    ept2 = k2 * B
    rpt = n_pad // NS
    d = y.shape[1]

    mesh = plsc.VectorSubcoreMesh(core_axis_name="c", subcore_axis_name="s")

    @functools.partial(
        pl.kernel,
        out_type=jax.ShapeDtypeStruct((NC, n_pad, d), jnp.float32),
        mesh=mesh,
        compiler_params=pltpu.CompilerParams(use_tc_tiling_on_sc=False),
        scratch_types=[
            pltpu.VMEM((ept2,), jnp.int32),
            pltpu.VMEM((k2, B), jnp.int32),
            pltpu.VMEM((B, d), jnp.float32),
            pltpu.VMEM((B, d), jnp.float32),
            pltpu.VMEM_SHARED((n_pad, d), jnp.float32),
            pltpu.SemaphoreType.DMA,
            pltpu.SemaphoreType.DMA,
        ],
    )
    def agg_kernel(y_hbm, src_hbm, dst_hbm, out_hbm,
                   srcv, dstv, rows0, rows1, acc_sh, sem0, sem1):
        c = lax.axis_index("c")
        s = lax.axis_index("s")
        w = c * NS + s

        def zrow(i, carry):
            for j in range(d // 16):
                rows0[i, pl.ds(j * 16, 16)] = jnp.zeros((16,), jnp.float32)
            return carry

        lax.fori_loop(0, B, zrow, 0)

        def zbody(j, carry):
            pltpu.async_copy(rows0, acc_sh.at[pl.ds(s * rpt + j * B, B)],
                             sem0)
            return carry

        lax.fori_loop(0, rpt // B, zbody, 0)

        pltpu.sync_copy(src_hbm.at[pl.ds(w * ept, ept2)], srcv)
        pltpu.sync_copy(dst_hbm.at[pl.ds(w * k, k2)], dstv)

        def zdrain(j, carry):
            pltpu.make_async_copy(
                rows0, acc_sh.at[pl.ds(s * rpt + j * B, B)], sem0).wait()
            return carry

        lax.fori_loop(0, rpt // B, zdrain, 0)
        plsc.subcore_barrier()

        for h in range(2):
            if h == 1:
                pltpu.sync_copy(src_hbm.at[pl.ds(w * ept + ept2, ept2)],
                                srcv)
                pltpu.sync_copy(dst_hbm.at[pl.ds(w * k + k2, k2)], dstv)
            pltpu.async_copy(y_hbm.at[srcv.at[pl.ds(0, B)]], rows0, sem0)

            def body(i, carry):
                j0 = 2 * i
                pltpu.async_copy(
                    y_hbm.at[srcv.at[pl.ds((j0 + 1) * B, B)]], rows1, sem1)
                pltpu.make_async_copy(
                    y_hbm.at[srcv.at[pl.ds(j0 * B, B)]], rows0, sem0).wait()
                pltpu.sync_copy(rows0, acc_sh.at[dstv.at[j0]], add=True)

                @pl.when(i + 1 < k2 // 2)
                def _():
                    pltpu.async_copy(
                        y_hbm.at[srcv.at[pl.ds((j0 + 2) * B, B)]], rows0,
                        sem0)

                pltpu.make_async_copy(
                    y_hbm.at[srcv.at[pl.ds((j0 + 1) * B, B)]], rows1,
                    sem1).wait()
                pltpu.sync_copy(rows1, acc_sh.at[dstv.at[j0 + 1]], add=True)
                return carry

            lax.fori_loop(0, k2 // 2, body, 0)

        plsc.subcore_barrier()
        pltpu.sync_copy(acc_sh.at[pl.ds(s * rpt, rpt)],
                        out_hbm.at[c, pl.ds(s * rpt, rpt)])

    return agg_kernel(y, src1d, dst2d)


def _tc_scale_matmul(x, w1, dega, degb, blk):
    n, d_in = x.shape
    d_out = w1.shape[1]
    grid = (n // blk,)

    def body(x_ref, w_ref, a_ref, b_ref, y_ref):
        dis = lax.rsqrt(1.0 + a_ref[...] + b_ref[...])
        y_ref[...] = jnp.dot(x_ref[...], w_ref[...],
                             preferred_element_type=jnp.float32) * dis

    return pl.pallas_call(
        body,
        grid=grid,
        in_specs=[
            pl.BlockSpec((blk, d_in), lambda i: (i, 0)),
            pl.BlockSpec((d_in, d_out), lambda i: (0, 0)),
            pl.BlockSpec((blk, 1), lambda i: (i, 0)),
            pl.BlockSpec((blk, 1), lambda i: (i, 0)),
        ],
        out_specs=pl.BlockSpec((blk, d_out), lambda i: (i, 0)),
        out_shape=jax.ShapeDtypeStruct((n, d_out), jnp.float32),
    )(x, w1, dega, degb)


def _tc_epilogue_matmul(acc, y1, dega, degb, b1, w2, blk):
    n, d = y1.shape
    d_out = w2.shape[1]
    grid = (n // blk,)

    def body(acc_ref, y1_ref, a_ref, b_ref, b1_ref, w_ref, y2_ref):
        dis = lax.rsqrt(1.0 + a_ref[...] + b_ref[...])
        h = (acc_ref[0] + acc_ref[1] + y1_ref[...]) * dis + b1_ref[...]
        h = jnp.maximum(h, 0.0)
        y2_ref[...] = jnp.dot(h, w_ref[...],
                              preferred_element_type=jnp.float32) * dis

    return pl.pallas_call(
        body,
        grid=grid,
        in_specs=[
            pl.BlockSpec((NC, blk, d), lambda i: (0, i, 0)),
            pl.BlockSpec((blk, d), lambda i: (i, 0)),
            pl.BlockSpec((blk, 1), lambda i: (i, 0)),
            pl.BlockSpec((blk, 1), lambda i: (i, 0)),
            pl.BlockSpec((1, d), lambda i: (0, 0)),
            pl.BlockSpec((d, d_out), lambda i: (0, 0)),
        ],
        out_specs=pl.BlockSpec((blk, d_out), lambda i: (i, 0)),
        out_shape=jax.ShapeDtypeStruct((n, d_out), jnp.float32),
    )(acc, y1, dega, degb, b1, w2)


def _tc_final(acc, y2, dega, degb, b2, blk):
    n, d = y2.shape
    grid = (n // blk,)

    def body(acc_ref, y2_ref, a_ref, b_ref, b2_ref, o_ref):
        dis = lax.rsqrt(1.0 + a_ref[...] + b_ref[...])
        o_ref[...] = (acc_ref[0] + acc_ref[1] + y2_ref[...]) * dis \
            + b2_ref[...]

    return pl.pallas_call(
        body,
        grid=grid,
        in_specs=[
            pl.BlockSpec((NC, blk, d), lambda i: (0, i, 0)),
            pl.BlockSpec((blk, d), lambda i: (i, 0)),
            pl.BlockSpec((blk, 1), lambda i: (i, 0)),
            pl.BlockSpec((blk, 1), lambda i: (i, 0)),
            pl.BlockSpec((1, d), lambda i: (0, 0)),
        ],
        out_specs=pl.BlockSpec((blk, d), lambda i: (i, 0)),
        out_shape=jax.ShapeDtypeStruct((n, d), jnp.float32),
    )(acc, y2, dega, degb, b2)


def kernel(x, edge_index, W1, b1, W2, b2):
    n, d_in = x.shape
    e = edge_index.shape[1]

    k = -(-e // (NW * B * 8)) * 8
    e_pad = NW * k * B
    n_pad = -(-(n + 1) // (B * NS)) * (B * NS)
    n_dummy = n_pad - n

    src = edge_index[0].astype(jnp.int32)
    dst = edge_index[1].astype(jnp.int32)
    npe = e_pad - e
    pad_idx = jnp.arange(npe, dtype=jnp.int32)
    src1d = jnp.concatenate([src, (pad_idx * 997) % n])
    dst2d = jnp.concatenate([dst, n + pad_idx % n_dummy]).reshape(NW * k, B)

    degp = _sc_degree(dst2d, n_pad, k)
    dega = degp[0].reshape(n_pad, 1)
    degb = degp[1].reshape(n_pad, 1)

    blk = 2000
    y1 = _tc_scale_matmul(x, W1, dega, degb, blk)
    acc1 = _sc_aggregate(y1, src1d, dst2d, n_pad, k)
    y2 = _tc_epilogue_matmul(acc1, y1, dega, degb, b1.reshape(1, -1), W2,
                             blk)
    acc2 = _sc_aggregate(y2, src1d, dst2d, n_pad, k)
    out = _tc_final(acc2, y2, dega, degb, b2.reshape(1, -1), blk)
    return out


# speedup vs baseline: 1.0359x; 1.0008x over previous
import functools

import jax
import jax.numpy as jnp
from jax import lax
from jax.experimental import pallas as pl
from jax.experimental.pallas import tpu as pltpu
from jax.experimental.pallas import tpu_sc as plsc

NC = 2
NS = 16
NW = NC * NS
B = 128


def _sc_degree(dst2d, n_pad, k):
    rpt = n_pad // NS

    mesh = plsc.VectorSubcoreMesh(core_axis_name="c", subcore_axis_name="s")

    @functools.partial(
        pl.kernel,
        out_type=jax.ShapeDtypeStruct((NC, n_pad), jnp.float32),
        mesh=mesh,
        compiler_params=pltpu.CompilerParams(use_tc_tiling_on_sc=False),
        scratch_types=[
            pltpu.VMEM((k, B), jnp.int32),
            pltpu.VMEM((B,), jnp.float32),
            pltpu.VMEM((B,), jnp.float32),
            pltpu.VMEM_SHARED((n_pad,), jnp.float32),
            pltpu.SemaphoreType.DMA,
        ],
    )
    def deg_kernel(dst_hbm, deg_out, dstv, onesv, zerov, deg_sh, dsem):
        c = lax.axis_index("c")
        s = lax.axis_index("s")
        w = c * NS + s
        for i in range(B // 16):
            onesv[pl.ds(i * 16, 16)] = jnp.full((16,), 1.0, jnp.float32)
            zerov[pl.ds(i * 16, 16)] = jnp.zeros((16,), jnp.float32)

        def zbody(j, carry):
            pltpu.async_copy(zerov, deg_sh.at[pl.ds(s * rpt + j * B, B)],
                             dsem)
            return carry

        lax.fori_loop(0, rpt // B, zbody, 0)
        pltpu.sync_copy(dst_hbm.at[pl.ds(w * k, k)], dstv)

        def zdrain(j, carry):
            pltpu.make_async_copy(
                zerov, deg_sh.at[pl.ds(s * rpt + j * B, B)], dsem).wait()
            return carry

        lax.fori_loop(0, rpt // B, zdrain, 0)
        plsc.subcore_barrier()

        def body(j, carry):
            pltpu.async_copy(onesv, deg_sh.at[dstv.at[j]], dsem, add=True)
            return carry

        lax.fori_loop(0, k, body, 0)

        def dbody(j, carry):
            pltpu.make_async_copy(onesv, deg_sh.at[dstv.at[j]], dsem).wait()
            return carry

        lax.fori_loop(0, k, dbody, 0)
        plsc.subcore_barrier()
        pltpu.sync_copy(deg_sh.at[pl.ds(s * rpt, rpt)],
                        deg_out.at[c, pl.ds(s * rpt, rpt)])

    return deg_kernel(dst2d)


def _sc_aggregate(y, src1d, dst2d, n_pad, k):
    ept = k * B
    k2 = k // 2
    ept2 = k2 * B
    rpt = n_pad // NS
    d = y.shape[1]

    mesh = plsc.VectorSubcoreMesh(core_axis_name="c", subcore_axis_name="s")

    @functools.partial(
        pl.kernel,
        out_type=jax.ShapeDtypeStruct((NC, n_pad, d), jnp.float32),
        mesh=mesh,
        compiler_params=pltpu.CompilerParams(use_tc_tiling_on_sc=False),
        scratch_types=[
            pltpu.VMEM((ept2,), jnp.int32),
            pltpu.VMEM((k2, B), jnp.int32),
            pltpu.VMEM((B, d), jnp.float32),
            pltpu.VMEM((B, d), jnp.float32),
            pltpu.VMEM_SHARED((n_pad, d), jnp.float32),
            pltpu.SemaphoreType.DMA,
            pltpu.SemaphoreType.DMA,
        ],
    )
    def agg_kernel(y_hbm, src_hbm, dst_hbm, out_hbm,
                   srcv, dstv, rows0, rows1, acc_sh, sem0, sem1):
        c = lax.axis_index("c")
        s = lax.axis_index("s")
        w = c * NS + s

        def zrow(i, carry):
            for j in range(d // 16):
                rows0[i, pl.ds(j * 16, 16)] = jnp.zeros((16,), jnp.float32)
            return carry

        lax.fori_loop(0, B, zrow, 0)

        def zbody(j, carry):
            pltpu.async_copy(rows0, acc_sh.at[pl.ds(s * rpt + j * B, B)],
                             sem0)
            return carry

        lax.fori_loop(0, rpt // B, zbody, 0)

        pltpu.sync_copy(src_hbm.at[pl.ds(w * ept, ept2)], srcv)
        pltpu.sync_copy(dst_hbm.at[pl.ds(w * k, k2)], dstv)

        def zdrain(j, carry):
            pltpu.make_async_copy(
                rows0, acc_sh.at[pl.ds(s * rpt + j * B, B)], sem0).wait()
            return carry

        lax.fori_loop(0, rpt // B, zdrain, 0)
        plsc.subcore_barrier()

        for h in range(2):
            if h == 1:
                pltpu.sync_copy(src_hbm.at[pl.ds(w * ept + ept2, ept2)],
                                srcv)
                pltpu.sync_copy(dst_hbm.at[pl.ds(w * k + k2, k2)], dstv)
            pltpu.async_copy(y_hbm.at[srcv.at[pl.ds(0, B)]], rows0, sem0)

            def body(i, carry):
                j0 = 2 * i
                pltpu.async_copy(
                    y_hbm.at[srcv.at[pl.ds((j0 + 1) * B, B)]], rows1, sem1)
                pltpu.make_async_copy(
                    y_hbm.at[srcv.at[pl.ds(j0 * B, B)]], rows0, sem0).wait()
                pltpu.sync_copy(rows0, acc_sh.at[dstv.at[j0]], add=True)

                @pl.when(i + 1 < k2 // 2)
                def _():
                    pltpu.async_copy(
                        y_hbm.at[srcv.at[pl.ds((j0 + 2) * B, B)]], rows0,
                        sem0)

                pltpu.make_async_copy(
                    y_hbm.at[srcv.at[pl.ds((j0 + 1) * B, B)]], rows1,
                    sem1).wait()
                pltpu.sync_copy(rows1, acc_sh.at[dstv.at[j0 + 1]], add=True)
                return carry

            lax.fori_loop(0, k2 // 2, body, 0)

        plsc.subcore_barrier()
        pltpu.sync_copy(acc_sh.at[pl.ds(s * rpt, rpt)],
                        out_hbm.at[c, pl.ds(s * rpt, rpt)])

    return agg_kernel(y, src1d, dst2d)


def _tc_scale_matmul(x, w1, dega, degb, blk):
    n, d_in = x.shape
    d_out = w1.shape[1]
    grid = (n // blk,)

    def body(x_ref, w_ref, a_ref, b_ref, y_ref):
        dis = lax.rsqrt(1.0 + a_ref[...] + b_ref[...])
        y_ref[...] = jnp.dot(x_ref[...], w_ref[...],
                             preferred_element_type=jnp.float32) * dis

    return pl.pallas_call(
        body,
        grid=grid,
        in_specs=[
            pl.BlockSpec((blk, d_in), lambda i: (i, 0)),
            pl.BlockSpec((d_in, d_out), lambda i: (0, 0)),
            pl.BlockSpec((blk, 1), lambda i: (i, 0)),
            pl.BlockSpec((blk, 1), lambda i: (i, 0)),
        ],
        out_specs=pl.BlockSpec((blk, d_out), lambda i: (i, 0)),
        out_shape=jax.ShapeDtypeStruct((n, d_out), jnp.float32),
    )(x, w1, dega, degb)


def _tc_epilogue_matmul(acc, y1, dega, degb, b1, w2, blk):
    n, d = y1.shape
    d_out = w2.shape[1]
    grid = (n // blk,)

    def body(acc_ref, y1_ref, a_ref, b_ref, b1_ref, w_ref, y2_ref):
        dis = lax.rsqrt(1.0 + a_ref[...] + b_ref[...])
        h = (acc_ref[0] + acc_ref[1] + y1_ref[...]) * dis + b1_ref[...]
        h = jnp.maximum(h, 0.0)
        y2_ref[...] = jnp.dot(h, w_ref[...],
                              preferred_element_type=jnp.float32) * dis

    return pl.pallas_call(
        body,
        grid=grid,
        in_specs=[
            pl.BlockSpec((NC, blk, d), lambda i: (0, i, 0)),
            pl.BlockSpec((blk, d), lambda i: (i, 0)),
            pl.BlockSpec((blk, 1), lambda i: (i, 0)),
            pl.BlockSpec((blk, 1), lambda i: (i, 0)),
            pl.BlockSpec((1, d), lambda i: (0, 0)),
            pl.BlockSpec((d, d_out), lambda i: (0, 0)),
        ],
        out_specs=pl.BlockSpec((blk, d_out), lambda i: (i, 0)),
        out_shape=jax.ShapeDtypeStruct((n, d_out), jnp.float32),
    )(acc, y1, dega, degb, b1, w2)


def _tc_final(acc, y2, dega, degb, b2, blk):
    n, d = y2.shape
    grid = (n // blk,)

    def body(acc_ref, y2_ref, a_ref, b_ref, b2_ref, o_ref):
        dis = lax.rsqrt(1.0 + a_ref[...] + b_ref[...])
        o_ref[...] = (acc_ref[0] + acc_ref[1] + y2_ref[...]) * dis \
            + b2_ref[...]

    return pl.pallas_call(
        body,
        grid=grid,
        in_specs=[
            pl.BlockSpec((NC, blk, d), lambda i: (0, i, 0)),
            pl.BlockSpec((blk, d), lambda i: (i, 0)),
            pl.BlockSpec((blk, 1), lambda i: (i, 0)),
            pl.BlockSpec((blk, 1), lambda i: (i, 0)),
            pl.BlockSpec((1, d), lambda i: (0, 0)),
        ],
        out_specs=pl.BlockSpec((blk, d), lambda i: (i, 0)),
        out_shape=jax.ShapeDtypeStruct((n, d), jnp.float32),
    )(acc, y2, dega, degb, b2)


def kernel(x, edge_index, W1, b1, W2, b2):
    n, d_in = x.shape
    e = edge_index.shape[1]

    k = -(-e // (NW * B * 8)) * 8
    e_pad = NW * k * B
    n_pad = -(-(n + 1) // (B * NS)) * (B * NS)
    n_dummy = n_pad - n

    src = edge_index[0].astype(jnp.int32)
    dst = edge_index[1].astype(jnp.int32)
    npe = e_pad - e
    pad_idx = jnp.arange(npe, dtype=jnp.int32)
    src1d = jnp.concatenate([src, (pad_idx * 997) % n])
    dst2d = jnp.concatenate([dst, n + pad_idx % n_dummy]).reshape(NW * k, B)

    degp = _sc_degree(dst2d, n_pad, k)
    dega = degp[0].reshape(n_pad, 1)
    degb = degp[1].reshape(n_pad, 1)

    blk = 2000
    y1 = _tc_scale_matmul(x, W1, dega, degb, blk)
    acc1 = _sc_aggregate(y1, src1d, dst2d, n_pad, k)
    y2 = _tc_epilogue_matmul(acc1, y1, dega, degb, b1.reshape(1, -1), W2,
                             blk)
    acc2 = _sc_aggregate(y2, src1d, dst2d, n_pad, k)
    out = _tc_final(acc2, y2, dega, degb, b2.reshape(1, -1), blk)
    return out
